# baseline (speedup 1.0000x reference)
import jax
import jax.numpy as jnp
from jax.experimental import pallas as pl
from jax.experimental.pallas import tpu as pltpu

_NT = 16384
_D = 2048
_E = 16
_BT = 1024
_STEPS = _NT // _BT
_NBUF = 4


def _gating_kernel(x_hbm, w_ref, gate_ref, idx_ref, loss_ref, bufs, sems):
    w = w_ref[...]
    for k in range(_NBUF):
        pltpu.make_async_copy(
            x_hbm.at[pl.ds(k * _BT, _BT), :], bufs.at[k], sems.at[k]
        ).start()
    psum = jnp.zeros((_E,), jnp.float32)
    csum = jnp.zeros((_E,), jnp.float32)
    for i in range(_STEPS):
        slot = i % _NBUF
        pltpu.make_async_copy(
            x_hbm.at[pl.ds(i * _BT, _BT), :], bufs.at[slot], sems.at[slot]
        ).wait()
        x = bufs[slot]
        logits = jax.lax.dot_general(
            x, w, (((1,), (1,)), ((), ())), preferred_element_type=jnp.float32
        )
        nxt = i + _NBUF
        if nxt < _STEPS:
            pltpu.make_async_copy(
                x_hbm.at[pl.ds(nxt * _BT, _BT), :], bufs.at[slot], sems.at[slot]
            ).start()
        iota = jax.lax.broadcasted_iota(jnp.int32, logits.shape, 1)
        m1 = jnp.max(logits, axis=1, keepdims=True)
        is1 = logits == m1
        idx1 = jnp.min(jnp.where(is1, iota, _E), axis=1, keepdims=True)
        masked = jnp.where(iota == idx1, -jnp.inf, logits)
        m2 = jnp.max(masked, axis=1, keepdims=True)
        idx2 = jnp.min(jnp.where(masked == m2, iota, _E), axis=1, keepdims=True)
        t = jnp.exp(m2 - m1)
        denom = 1.0 + t
        gate_ref[pl.ds(i * _BT, _BT), :] = jnp.concatenate(
            [1.0 / denom, t / denom], axis=1
        )
        idx_ref[pl.ds(i * _BT, _BT), :] = jnp.concatenate([idx1, idx2], axis=1)
        e = jnp.exp(logits - m1)
        p = e / jnp.sum(e, axis=1, keepdims=True)
        psum = psum + jnp.sum(p, axis=0)
        csum = csum + jnp.sum((iota == idx1).astype(jnp.float32), axis=0)
    loss = _E * jnp.sum(psum * csum, keepdims=True) / (_NT * _NT)
    loss_ref[...] = loss.reshape(1, 1)


def kernel(x, W):
    gate, idx, loss = pl.pallas_call(
        _gating_kernel,
        in_specs=[
            pl.BlockSpec(memory_space=pl.ANY),
            pl.BlockSpec(memory_space=pltpu.MemorySpace.VMEM),
        ],
        out_specs=[
            pl.BlockSpec(memory_space=pltpu.MemorySpace.VMEM),
            pl.BlockSpec(memory_space=pltpu.MemorySpace.VMEM),
            pl.BlockSpec(memory_space=pltpu.MemorySpace.VMEM),
        ],
        out_shape=[
            jax.ShapeDtypeStruct((_NT, 2), jnp.float32),
            jax.ShapeDtypeStruct((_NT, 2), jnp.int32),
            jax.ShapeDtypeStruct((1, 1), jnp.float32),
        ],
        scratch_shapes=[
            pltpu.VMEM((_NBUF, _BT, _D), jnp.float32),
            pltpu.SemaphoreType.DMA((_NBUF,)),
        ],
        compiler_params=pltpu.CompilerParams(
            vmem_limit_bytes=100 * 1024 * 1024
        ),
    )(x, W)
    return gate, idx, loss[0, 0]


# speedup vs baseline: 1.8623x; 1.8623x over previous
import jax
import jax.numpy as jnp
from jax.experimental import pallas as pl
from jax.experimental.pallas import tpu as pltpu

_NT = 16384
_D = 2048
_E = 16
_BT = 1024
_STEPS = _NT // _BT


def _gating_kernel(x_ref, w_ref, gate_ref, idx_ref, loss_ref, acc_ref):
    step = pl.program_id(0)
    x = x_ref[...]
    w = w_ref[...]
    lt = jax.lax.dot_general(
        w, x, (((1,), (1,)), ((), ())), preferred_element_type=jnp.float32
    )
    iota = jax.lax.broadcasted_iota(jnp.int32, lt.shape, 0)
    m1 = jnp.max(lt, axis=0, keepdims=True)
    is1 = lt == m1
    idx1 = jnp.min(jnp.where(is1, iota, _E), axis=0, keepdims=True)
    masked = jnp.where(iota == idx1, -jnp.inf, lt)
    m2 = jnp.max(masked, axis=0, keepdims=True)
    idx2 = jnp.min(jnp.where(masked == m2, iota, _E), axis=0, keepdims=True)
    t = jnp.exp(m2 - m1)
    denom = 1.0 + t
    gate_ref[...] = jnp.concatenate([1.0 / denom, t / denom], axis=0)
    idx_ref[...] = jnp.concatenate([idx1, idx2], axis=0)
    e = jnp.exp(lt - m1)
    p = e / jnp.sum(e, axis=0, keepdims=True)
    psum = jnp.sum(p, axis=1)
    csum = jnp.sum((iota == idx1).astype(jnp.float32), axis=1)
    part = jnp.stack([psum, csum])

    @pl.when(step == 0)
    def _init():
        acc_ref[...] = part

    @pl.when(step != 0)
    def _accum():
        acc_ref[...] += part

    @pl.when(step == _STEPS - 1)
    def _finalize():
        acc = acc_ref[...]
        loss = _E * jnp.sum(acc[0] * acc[1], keepdims=True) / (_NT * _NT)
        loss_ref[...] = loss.reshape(1, 1)


def kernel(x, W):
    gate_t, idx_t, loss = pl.pallas_call(
        _gating_kernel,
        grid=(_STEPS,),
        in_specs=[
            pl.BlockSpec((_BT, _D), lambda i: (i, 0)),
            pl.BlockSpec((_E, _D), lambda i: (0, 0)),
        ],
        out_specs=[
            pl.BlockSpec((2, _BT), lambda i: (0, i)),
            pl.BlockSpec((2, _BT), lambda i: (0, i)),
            pl.BlockSpec((1, 1), lambda i: (0, 0)),
        ],
        out_shape=[
            jax.ShapeDtypeStruct((2, _NT), jnp.float32),
            jax.ShapeDtypeStruct((2, _NT), jnp.int32),
            jax.ShapeDtypeStruct((1, 1), jnp.float32),
        ],
        scratch_shapes=[pltpu.VMEM((2, _E), jnp.float32)],
        compiler_params=pltpu.CompilerParams(
            vmem_limit_bytes=100 * 1024 * 1024
        ),
    )(x, W)
    return gate_t.T, idx_t.T, loss[0, 0]
